# baseline (speedup 1.0000x reference)
import jax
import jax.numpy as jnp
from jax import lax
from jax.experimental import pallas as pl
from jax.experimental.pallas import tpu as pltpu
from jax.experimental.pallas import tpu_sc as plsc

_N = 10000
_E = 160000
_D = 256
_L = 16
_NK = _D // _L
_NC = 2
_NS = 16
_NW = _NC * _NS
_BLK = 128
_OB = 64
_HW = 128
_NPAD = ((_N + _HW - 1) // _HW) * _HW


def _body(hw_ref, h_ref, vi_ref, vj_ref, prm_ref, out_ref,
          vi_va, vi_vb, vj_va, vj_vb, hvj_a, hvj_b,
          outbuf, hww, prm_v, s0, s1):
    c = lax.axis_index("c")
    s = lax.axis_index("s")
    wid = s * _NC + c

    pltpu.sync_copy(prm_ref, prm_v)

    def sload(ref, i):
        return ref[pl.ds(i, _L)][0]

    e_lo = sload(prm_v, wid)
    e_hi = sload(prm_v, wid + 1)
    zb = sload(prm_v, _NW + 1 + wid)
    zb1 = sload(prm_v, _NW + 1 + wid + 1)

    zero = jnp.zeros((_L,), jnp.float32)

    def memset_buf():
        def mrow(r, carry):
            for k in range(_NK):
                outbuf[r, pl.ds(k * _L, _L)] = zero
            return carry
        lax.fori_loop(0, _OB, mrow, 0)

    memset_buf()

    def full_flush(win):
        pltpu.sync_copy(outbuf, out_ref.at[pl.ds(pl.multiple_of(win, 8), _OB)])
        memset_buf()

    def write_row(node, win, esum, accs):
        def advc(w):
            return node >= w + _OB

        def advb(w):
            full_flush(w)
            return w + _OB

        win = lax.while_loop(advc, advb, win)
        r = node - win
        rv = 1.0 / esum
        for k in range(_NK):
            outbuf[r, pl.ds(k * _L, _L)] = accs[k] * rv
        return win

    lanes = lax.iota(jnp.int32, _L)

    e_lo8 = jnp.bitwise_and(e_lo, jnp.int32(-8))
    span = e_hi - e_lo8
    nblk = (span + (_BLK - 1)) // _BLK

    def base_of(b):
        return pl.multiple_of(e_lo8 + b * _BLK, 8)

    bufs = ((vi_va, vj_va, hvj_a, s0), (vi_vb, vj_vb, hvj_b, s1))

    def load_and_issue(b, buf):
        vi_v, vj_v, hvj_v, sem = bufs[buf]
        base = base_of(b)
        pltpu.sync_copy(vi_ref.at[pl.ds(base, _BLK)],
                        vi_v.at[pl.ds(0, _BLK)])
        pltpu.sync_copy(vj_ref.at[pl.ds(base, _BLK)], vj_v)
        half = _BLK // 2
        pltpu.async_copy(h_ref.at[vj_v.at[pl.ds(0, half)]],
                         hvj_v.at[pl.ds(0, half)], sem)
        pltpu.async_copy(h_ref.at[vj_v.at[pl.ds(half, half)]],
                         hvj_v.at[pl.ds(half, half)], sem)

    def compute(b, buf, C):
        vi_v, vj_v, hvj_v, sem = bufs[buf]
        half = _BLK // 2
        pltpu.make_async_copy(h_ref.at[vj_v.at[pl.ds(0, half)]],
                              hvj_v.at[pl.ds(0, half)], sem).wait()
        pltpu.make_async_copy(h_ref.at[vj_v.at[pl.ds(half, half)]],
                              hvj_v.at[pl.ds(half, half)], sem).wait()
        base = base_of(b)

        def run_cond(S):
            return S[0] < _BLK

        def run_body(S):
            i = S[0]
            C = S[1:]
            node = vi_v[pl.ds(i, _L)][0]
            changed = node != C[0]

            def on_changed(C2):
                pos = jnp.max(C2[3]) > 0.0

                def fl(C3):
                    w3 = write_row(C3[0], C3[1], C3[3], C3[20:36])
                    return (C3[0], w3) + C3[2:]

                C2 = lax.cond(pos, fl, lambda x: x, C2)
                nwb = jnp.bitwise_and(node, jnp.int32(-_HW))

                @pl.when(nwb != C2[2])
                def _():
                    pltpu.sync_copy(
                        hw_ref.at[pl.ds(pl.multiple_of(nwb, 8), _HW)], hww)

                r = jnp.bitwise_and(node, jnp.int32(_HW - 1))
                svi = tuple(hww[r, pl.ds(k * _L, _L)] for k in range(_NK))
                return (node, C2[1], nwb, zero) + svi + (zero,) * _NK

            C = lax.cond(changed, on_changed, lambda x: x, C)
            svi = C[4:20]

            def f_cond(T):
                return T[1] >= _L

            def f_body(T):
                j = T[0]
                v = vi_v[pl.ds(j, _L)]
                m = jnp.logical_or(v != node, lanes >= (_BLK - j))
                f = plsc.all_reduce_ffs(m)[0]
                return (j + f, f)

            j = lax.while_loop(f_cond, f_body, (i, jnp.int32(_L)))[0]

            owned = jnp.logical_and(base + i >= e_lo, base + i < e_hi)

            def accum(A):
                @plsc.parallel_loop(i, j, unroll=3, carry=A)
                def inner(ii, A2):
                    esum, accs = A2[0], A2[1:]
                    hj = [hvj_v[ii, pl.ds(k * _L, _L)] for k in range(_NK)]
                    p = [svi[k] * hj[k] for k in range(_NK)]
                    while len(p) > 1:
                        p = [p[2 * m] + p[2 * m + 1]
                             for m in range(len(p) // 2)]
                    logit = jnp.sum(p[0])
                    pv = jnp.exp(jnp.full((_L,), logit, jnp.float32))
                    esum = esum + pv
                    accs = tuple(accs[k] + pv * hj[k] for k in range(_NK))
                    return (esum,) + accs

                return inner

            A = (C[3],) + C[20:36]
            A = lax.cond(owned, accum, lambda x: x, A)
            return (j,) + C[:3] + (A[0],) + svi + A[1:]

        S = lax.while_loop(run_cond, run_body, (jnp.int32(0),) + C)
        return S[1:]

    @pl.when(nblk > 0)
    def _():
        load_and_issue(0, 0)

    def blk2(b2, C):
        b = b2 * 2

        @pl.when(b + 1 < nblk)
        def _():
            load_and_issue(b + 1, 1)

        C = compute(b, 0, C)

        @pl.when(b + 2 < nblk)
        def _():
            load_and_issue(b + 2, 0)

        C = lax.cond(b + 1 < nblk,
                     lambda CC: compute(b + 1, 1, CC),
                     lambda CC: CC, C)
        return C

    C0 = ((jnp.int32(-1), zb, jnp.int32(-(1 << 30)), zero)
          + (zero,) * _NK + (zero,) * _NK)
    C = lax.fori_loop(0, (nblk + 1) // 2, blk2, C0)

    pos = jnp.max(C[3]) > 0.0
    win = lax.cond(pos,
                   lambda w: write_row(C[0], w, C[3], C[20:36]),
                   lambda w: w, C[1])

    def swc(w):
        return w + _OB <= zb1

    def swb(w):
        full_flush(w)
        return w + _OB

    win = lax.while_loop(swc, swb, win)
    rem = zb1 - win
    off = jnp.int32(0)
    for sz in (32, 16, 8):
        p = jnp.bitwise_and(rem, sz) != 0

        @pl.when(p)
        def _(off=off, sz=sz):
            pltpu.sync_copy(outbuf.at[pl.ds(pl.multiple_of(off, 8), sz)],
                            out_ref.at[pl.ds(pl.multiple_of(win + off, 8), sz)])

        off = jnp.where(p, off + sz, off)


def kernel(hidden, selected_edges, score_weight, bias):
    del bias
    h = hidden[0]
    vi = selected_edges[:, 1]
    vj = selected_edges[:, 2]
    hw = jnp.concatenate(
        [h * score_weight[None, :],
         jnp.zeros((_NPAD - _N, _D), jnp.float32)])

    raw = jnp.arange(_NW, dtype=jnp.int32) * (_E // _NW)
    nbv = jnp.bitwise_and(vi[raw], -8)
    bv = jnp.searchsorted(vi, nbv, side="left").astype(jnp.int32)
    bfull = jnp.concatenate([bv, jnp.array([_E], jnp.int32)])
    zbv = jnp.concatenate([jnp.array([0], jnp.int32), nbv[1:],
                           jnp.array([_N], jnp.int32)])
    prm = jnp.concatenate([bfull, zbv,
                           jnp.zeros((96 - 2 * (_NW + 1),), jnp.int32)])
    vi_p = jnp.concatenate([vi, jnp.full((_BLK,), _N, jnp.int32)])
    vj_p = jnp.concatenate([vj, jnp.zeros((_BLK,), jnp.int32)])

    mesh = plsc.VectorSubcoreMesh(core_axis_name="c", subcore_axis_name="s",
                                  num_cores=_NC, num_subcores=_NS)
    fn = pl.kernel(
        _body,
        out_type=jax.ShapeDtypeStruct((_N, _D), jnp.float32),
        mesh=mesh,
        compiler_params=pltpu.CompilerParams(needs_layout_passes=False),
        scratch_types=[
            pltpu.VMEM((_BLK + _L,), jnp.int32),
            pltpu.VMEM((_BLK + _L,), jnp.int32),
            pltpu.VMEM((_BLK,), jnp.int32),
            pltpu.VMEM((_BLK,), jnp.int32),
            pltpu.VMEM((_BLK, _D), jnp.float32),
            pltpu.VMEM((_BLK, _D), jnp.float32),
            pltpu.VMEM((_OB, _D), jnp.float32),
            pltpu.VMEM((_HW, _D), jnp.float32),
            pltpu.VMEM((96,), jnp.int32),
            pltpu.SemaphoreType.DMA,
            pltpu.SemaphoreType.DMA,
        ],
    )
    out = fn(hw, h, vi_p, vj_p, prm)
    return out[None]


# speedup vs baseline: 1.0591x; 1.0591x over previous
import jax
import jax.numpy as jnp
from jax import lax
from jax.experimental import pallas as pl
from jax.experimental.pallas import tpu as pltpu
from jax.experimental.pallas import tpu_sc as plsc

_N = 10000
_E = 160000
_D = 256
_L = 16
_NK = _D // _L
_NC = 2
_NS = 16
_NW = _NC * _NS
_BLK = 128
_OB = 64
_HW = 128
_NPAD = ((_N + _HW - 1) // _HW) * _HW


def _body(hw_ref, h_ref, vi_ref, vj_ref, prm_ref, out_ref,
          vi_va, vi_vb, vj_va, vj_vb, hvj_a, hvj_b,
          outbuf, hww, prm_v, s0, s1):
    c = lax.axis_index("c")
    s = lax.axis_index("s")
    wid = s * _NC + c

    pltpu.sync_copy(prm_ref, prm_v)

    def sload(ref, i):
        return ref[pl.ds(i, _L)][0]

    e_lo = sload(prm_v, wid)
    e_hi = sload(prm_v, wid + 1)
    zb = sload(prm_v, _NW + 1 + wid)
    zb1 = sload(prm_v, _NW + 1 + wid + 1)

    zero = jnp.zeros((_L,), jnp.float32)

    def memset_buf():
        def mrow(r, carry):
            for k in range(_NK):
                outbuf[r, pl.ds(k * _L, _L)] = zero
            return carry
        lax.fori_loop(0, _OB, mrow, 0)

    memset_buf()

    def full_flush(win):
        pltpu.sync_copy(outbuf, out_ref.at[pl.ds(pl.multiple_of(win, 8), _OB)])
        memset_buf()

    def write_row(node, win, esum, accs):
        def advc(w):
            return node >= w + _OB

        def advb(w):
            full_flush(w)
            return w + _OB

        win = lax.while_loop(advc, advb, win)
        r = node - win
        rv = 1.0 / esum
        for k in range(_NK):
            outbuf[r, pl.ds(k * _L, _L)] = accs[k] * rv
        return win

    lanes = lax.iota(jnp.int32, _L)

    e_lo8 = jnp.bitwise_and(e_lo, jnp.int32(-8))
    span = e_hi - e_lo8
    nblk = (span + (_BLK - 1)) // _BLK

    def base_of(b):
        return pl.multiple_of(e_lo8 + b * _BLK, 8)

    bufs = ((vi_va, vj_va, hvj_a, s0), (vi_vb, vj_vb, hvj_b, s1))

    def load_and_issue(b, buf):
        vi_v, vj_v, hvj_v, sem = bufs[buf]
        base = base_of(b)
        pltpu.sync_copy(vi_ref.at[pl.ds(base, _BLK)],
                        vi_v.at[pl.ds(0, _BLK)])
        pltpu.sync_copy(vj_ref.at[pl.ds(base, _BLK)], vj_v)
        half = _BLK // 2
        pltpu.async_copy(h_ref.at[vj_v.at[pl.ds(0, half)]],
                         hvj_v.at[pl.ds(0, half)], sem)
        pltpu.async_copy(h_ref.at[vj_v.at[pl.ds(half, half)]],
                         hvj_v.at[pl.ds(half, half)], sem)

    def compute(b, buf, C):
        vi_v, vj_v, hvj_v, sem = bufs[buf]
        half = _BLK // 2
        pltpu.make_async_copy(h_ref.at[vj_v.at[pl.ds(0, half)]],
                              hvj_v.at[pl.ds(0, half)], sem).wait()
        pltpu.make_async_copy(h_ref.at[vj_v.at[pl.ds(half, half)]],
                              hvj_v.at[pl.ds(half, half)], sem).wait()
        base = base_of(b)

        def run_cond(S):
            return S[0] < _BLK

        def run_body(S):
            i = S[0]
            C = S[1:]
            node = vi_v[pl.ds(i, _L)][0]
            changed = node != C[0]

            def on_changed(C2):
                pos = jnp.max(C2[3]) > 0.0

                def fl(C3):
                    w3 = write_row(C3[0], C3[1], C3[3], C3[20:36])
                    return (C3[0], w3) + C3[2:]

                C2 = lax.cond(pos, fl, lambda x: x, C2)
                nwb = jnp.bitwise_and(node, jnp.int32(-_HW))

                @pl.when(nwb != C2[2])
                def _():
                    pltpu.sync_copy(
                        hw_ref.at[pl.ds(pl.multiple_of(nwb, 8), _HW)], hww)

                r = jnp.bitwise_and(node, jnp.int32(_HW - 1))
                svi = tuple(hww[r, pl.ds(k * _L, _L)] for k in range(_NK))
                return (node, C2[1], nwb, zero) + svi + (zero,) * _NK

            C = lax.cond(changed, on_changed, lambda x: x, C)
            svi = C[4:20]

            def f_cond(T):
                return T[1] >= _L

            def f_body(T):
                j = T[0]
                v = vi_v[pl.ds(j, _L)]
                m = jnp.logical_or(v != node, lanes >= (_BLK - j))
                f = plsc.all_reduce_ffs(m)[0]
                return (j + f, f)

            j = lax.while_loop(f_cond, f_body, (i, jnp.int32(_L)))[0]

            owned = jnp.logical_and(base + i >= e_lo, base + i < e_hi)

            def accum(A):
                @plsc.parallel_loop(i, j, unroll=2, carry=A)
                def inner(ii, A2):
                    esum, accs = A2[0], A2[1:]
                    hj = [hvj_v[ii, pl.ds(k * _L, _L)] for k in range(_NK)]
                    p = [svi[k] * hj[k] for k in range(_NK)]
                    while len(p) > 1:
                        p = [p[2 * m] + p[2 * m + 1]
                             for m in range(len(p) // 2)]
                    logit = jnp.sum(p[0])
                    pv = jnp.exp(jnp.full((_L,), logit, jnp.float32))
                    esum = esum + pv
                    accs = tuple(accs[k] + pv * hj[k] for k in range(_NK))
                    return (esum,) + accs

                return inner

            A = (C[3],) + C[20:36]
            A = lax.cond(owned, accum, lambda x: x, A)
            return (j,) + C[:3] + (A[0],) + svi + A[1:]

        S = lax.while_loop(run_cond, run_body, (jnp.int32(0),) + C)
        return S[1:]

    @pl.when(nblk > 0)
    def _():
        load_and_issue(0, 0)

    def blk2(b2, C):
        b = b2 * 2

        @pl.when(b + 1 < nblk)
        def _():
            load_and_issue(b + 1, 1)

        C = compute(b, 0, C)

        @pl.when(b + 2 < nblk)
        def _():
            load_and_issue(b + 2, 0)

        C = lax.cond(b + 1 < nblk,
                     lambda CC: compute(b + 1, 1, CC),
                     lambda CC: CC, C)
        return C

    C0 = ((jnp.int32(-1), zb, jnp.int32(-(1 << 30)), zero)
          + (zero,) * _NK + (zero,) * _NK)
    C = lax.fori_loop(0, (nblk + 1) // 2, blk2, C0)

    pos = jnp.max(C[3]) > 0.0
    win = lax.cond(pos,
                   lambda w: write_row(C[0], w, C[3], C[20:36]),
                   lambda w: w, C[1])

    def swc(w):
        return w + _OB <= zb1

    def swb(w):
        full_flush(w)
        return w + _OB

    win = lax.while_loop(swc, swb, win)
    rem = zb1 - win
    off = jnp.int32(0)
    for sz in (32, 16, 8):
        p = jnp.bitwise_and(rem, sz) != 0

        @pl.when(p)
        def _(off=off, sz=sz):
            pltpu.sync_copy(outbuf.at[pl.ds(pl.multiple_of(off, 8), sz)],
                            out_ref.at[pl.ds(pl.multiple_of(win + off, 8), sz)])

        off = jnp.where(p, off + sz, off)


def kernel(hidden, selected_edges, score_weight, bias):
    del bias
    h = hidden[0]
    vi = selected_edges[:, 1]
    vj = selected_edges[:, 2]
    hw = jnp.concatenate(
        [h * score_weight[None, :],
         jnp.zeros((_NPAD - _N, _D), jnp.float32)])

    raw = jnp.arange(_NW, dtype=jnp.int32) * (_E // _NW)
    nbv = jnp.bitwise_and(vi[raw], -8)
    bv = jnp.searchsorted(vi, nbv, side="left").astype(jnp.int32)
    bfull = jnp.concatenate([bv, jnp.array([_E], jnp.int32)])
    zbv = jnp.concatenate([jnp.array([0], jnp.int32), nbv[1:],
                           jnp.array([_N], jnp.int32)])
    prm = jnp.concatenate([bfull, zbv,
                           jnp.zeros((96 - 2 * (_NW + 1),), jnp.int32)])
    vi_p = jnp.concatenate([vi, jnp.full((_BLK,), _N, jnp.int32)])
    vj_p = jnp.concatenate([vj, jnp.zeros((_BLK,), jnp.int32)])

    mesh = plsc.VectorSubcoreMesh(core_axis_name="c", subcore_axis_name="s",
                                  num_cores=_NC, num_subcores=_NS)
    fn = pl.kernel(
        _body,
        out_type=jax.ShapeDtypeStruct((_N, _D), jnp.float32),
        mesh=mesh,
        compiler_params=pltpu.CompilerParams(needs_layout_passes=False),
        scratch_types=[
            pltpu.VMEM((_BLK + _L,), jnp.int32),
            pltpu.VMEM((_BLK + _L,), jnp.int32),
            pltpu.VMEM((_BLK,), jnp.int32),
            pltpu.VMEM((_BLK,), jnp.int32),
            pltpu.VMEM((_BLK, _D), jnp.float32),
            pltpu.VMEM((_BLK, _D), jnp.float32),
            pltpu.VMEM((_OB, _D), jnp.float32),
            pltpu.VMEM((_HW, _D), jnp.float32),
            pltpu.VMEM((96,), jnp.int32),
            pltpu.SemaphoreType.DMA,
            pltpu.SemaphoreType.DMA,
        ],
    )
    out = fn(hw, h, vi_p, vj_p, prm)
    return out[None]
